# Initial kernel scaffold; baseline (speedup 1.0000x reference)
import jax
import jax.numpy as jnp
from jax.experimental import pallas as pl


def kernel(theta, phi, grid):
    raise NotImplementedError("write your pallas kernel here")


# speedup vs baseline: 1267.3318x; 1267.3318x over previous
import functools

import jax
import jax.numpy as jnp
import numpy as np
from jax import lax
from jax.experimental import pallas as pl
from jax.experimental.pallas import tpu as pltpu
from jax.experimental.pallas import tpu_sc as plsc

THETA_RES = 16
PHI_RES = 8
MAX_SCALE_LOG = 0.3
N = 16777216

NC = 2
NS = 16
NW = NC * NS
B_PER_W = N // NW
CHUNK = 16384
NCHUNK = B_PER_W // CHUNK
LANES = 16
NBUF = 2

C_T = np.float32(np.float64(THETA_RES) / (2.0 * np.pi))
C_P = np.float32(np.float64(PHI_RES) / np.pi)


def _sc_body(theta_hbm, phi_hbm, grid_hbm, out_hbm,
             tbuf0, tbuf1, pbuf0, pbuf1, obuf0, obuf1, gbuf, table,
             lsem0, lsem1, osem0, osem1):
    wid = lax.axis_index("s") * NC + lax.axis_index("c")
    base = wid * B_PER_W
    tbufs = (tbuf0, tbuf1)
    pbufs = (pbuf0, pbuf1)
    obufs = (obuf0, obuf1)
    lsems = (lsem0, lsem1)
    osems = (osem0, osem1)

    pltpu.sync_copy(grid_hbm, gbuf)

    @pl.loop(0, 8)
    def _table(v):
        g = gbuf[pl.ds(v * LANES, LANES)]
        table[pl.ds(v * LANES, LANES)] = jnp.exp(
            jnp.clip(g, -MAX_SCALE_LOG, MAX_SCALE_LOG))

    def start_load(g, b):
        off = base + g * CHUNK
        pltpu.async_copy(theta_hbm.at[pl.ds(off, CHUNK)], tbufs[b], lsems[b])
        pltpu.async_copy(phi_hbm.at[pl.ds(off, CHUNK)], pbufs[b], lsems[b])

    def wait_load(b):
        pltpu.make_async_copy(
            theta_hbm.at[pl.ds(0, CHUNK)], tbufs[b], lsems[b]).wait()
        pltpu.make_async_copy(
            phi_hbm.at[pl.ds(0, CHUNK)], pbufs[b], lsems[b]).wait()

    def start_store(g, b):
        off = base + g * CHUNK
        pltpu.async_copy(obufs[b], out_hbm.at[pl.ds(off, CHUNK)], osems[b])

    def wait_store(b):
        pltpu.make_async_copy(
            obufs[b], out_hbm.at[pl.ds(0, CHUNK)], osems[b]).wait()

    start_load(0, 0)
    start_load(1, 1)

    @pl.loop(0, NCHUNK // NBUF)
    def _step(s):
        for b in range(NBUF):
            g = s * NBUF + b
            wait_load(b)

            @pl.when(s > 0)
            def _():
                wait_store(b)

            tb = tbufs[b]
            pb = pbufs[b]
            ob = obufs[b]

            @plsc.parallel_loop(0, CHUNK // LANES, unroll=8)
            def _vec(i):
                sl = pl.ds(i * LANES, LANES)
                t = tb[sl]
                p = pb[sl]
                ti = (t * C_T).astype(jnp.int32)
                pi = (p * C_P).astype(jnp.int32)
                ti = jnp.minimum(ti, THETA_RES - 1)
                pi = jnp.minimum(pi, PHI_RES - 1)
                idx = ti * PHI_RES + pi
                ob[sl] = plsc.load_gather(table, [idx])

            start_store(g, b)

            @pl.when(g + NBUF < NCHUNK)
            def _():
                start_load(g + NBUF, b)

    for b in range(NBUF):
        wait_store(b)


@jax.jit
def _sc_call(theta, phi, gridf):
    mesh = plsc.VectorSubcoreMesh(core_axis_name="c", subcore_axis_name="s")
    return pl.kernel(
        _sc_body,
        out_type=jax.ShapeDtypeStruct((N,), jnp.float32),
        mesh=mesh,
        scratch_types=[
            pltpu.VMEM((CHUNK,), jnp.float32),
            pltpu.VMEM((CHUNK,), jnp.float32),
            pltpu.VMEM((CHUNK,), jnp.float32),
            pltpu.VMEM((CHUNK,), jnp.float32),
            pltpu.VMEM((CHUNK,), jnp.float32),
            pltpu.VMEM((CHUNK,), jnp.float32),
            pltpu.VMEM((THETA_RES * PHI_RES,), jnp.float32),
            pltpu.VMEM((THETA_RES * PHI_RES,), jnp.float32),
            pltpu.SemaphoreType.DMA,
            pltpu.SemaphoreType.DMA,
            pltpu.SemaphoreType.DMA,
            pltpu.SemaphoreType.DMA,
        ],
        compiler_params=pltpu.CompilerParams(needs_layout_passes=False),
    )(theta, phi, gridf)


def kernel(theta, phi, grid):
    return _sc_call(theta, phi, grid.reshape(-1))
